# Initial kernel scaffold; baseline (speedup 1.0000x reference)
import jax
import jax.numpy as jnp
from jax.experimental import pallas as pl


def kernel(qos, item_avg, item_sim_agg, user_id, item_id, time_id):
    raise NotImplementedError("write your pallas kernel here")


# speedup vs baseline: 5.3073x; 5.3073x over previous
import jax
import jax.numpy as jnp
from jax import lax
from jax.experimental import pallas as pl
from jax.experimental.pallas import tpu as pltpu
from jax.experimental.pallas import tpu_sc as plsc

T, U, I = 16, 142, 4500
IP = 4608
K = 50
B = 4096
NW = 32
QPW = B // NW
G = 8
NCHUNK = QPW // G
L = 16
NB = IP // L
CANDCAP = 4608
MAXROUNDS = 24

_i32 = jnp.int32
_f32 = jnp.float32


def _iota():
  return lax.iota(_i32, L)


def _digit(v, base, scale):
  dv = jnp.clip((v - base) * scale, 0.0, 31.0)
  return dv.astype(_i32)


def _sc_body(sim_hbm, qos_hbm, avg_hbm, avgflat_hbm, rq_hbm, rs_hbm, ra_hbm,
             fia_hbm, out_hbm, simb, qosb, avgb, idxq, idxs, idxa, fi, avgi,
             preds, candv, candd, histc, hists, histn, sem_s, sem_q, sem_a,
             sem_f):
  wid = lax.axis_index("s") * 2 + lax.axis_index("c")
  iot = _iota()
  zf = jnp.zeros((L,), _f32)
  zi = jnp.zeros((L,), _i32)
  ones_i = jnp.ones((L,), _i32)

  pltpu.sync_copy(rq_hbm.at[wid], idxq)
  pltpu.sync_copy(rs_hbm.at[wid], idxs)
  pltpu.sync_copy(ra_hbm.at[wid], idxa)
  pltpu.sync_copy(fia_hbm.at[wid], fi)
  pltpu.async_copy(avgflat_hbm.at[fi], avgi, sem_f).wait()

  for h, z in ((histc, zi), (hists, zf), (histn, zf)):
    for j in range(512 // L):
      h[pl.ds(j * L, L)] = z

  def fold_hists():
    c0 = zi
    c1 = zi
    s0 = zf
    s1 = zf
    n0 = zf
    n1 = zf
    for lane in range(L):
      o = lane * 32
      c0 = c0 + histc[pl.ds(o, L)]
      c1 = c1 + histc[pl.ds(o + L, L)]
      s0 = s0 + hists[pl.ds(o, L)]
      s1 = s1 + hists[pl.ds(o + L, L)]
      n0 = n0 + histn[pl.ds(o, L)]
      n1 = n1 + histn[pl.ds(o + L, L)]
      histc[pl.ds(o, L)] = zi
      histc[pl.ds(o + L, L)] = zi
      hists[pl.ds(o, L)] = zf
      hists[pl.ds(o + L, L)] = zf
      histn[pl.ds(o, L)] = zf
      histn[pl.ds(o + L, L)] = zf
    return c0, c1, s0, s1, n0, n1

  def pick_bin(c0, c1, s0, s1, n0, n1, kk):
    t1 = jnp.sum(c1)
    rc0 = lax.rev(plsc.cumsum(lax.rev(c0, (0,))), (0,)) + t1
    rc1 = lax.rev(plsc.cumsum(lax.rev(c1, (0,))), (0,))
    b0 = jnp.max(jnp.where(rc0 >= kk, iot, -1))
    b1 = jnp.max(jnp.where(rc1 >= kk, iot + L, -1))
    bstar = jnp.maximum(b0, b1)
    gt0 = iot > bstar
    gt1 = (iot + L) > bstar
    c_gt = jnp.sum(jnp.where(gt0, c0, 0)) + jnp.sum(jnp.where(gt1, c1, 0))
    s_d = jnp.sum(jnp.where(gt0, s0, 0.0)) + jnp.sum(jnp.where(gt1, s1, 0.0))
    n_d = jnp.sum(jnp.where(gt0, n0, 0.0)) + jnp.sum(jnp.where(gt1, n1, 0.0))
    eq0 = iot == bstar
    eq1 = (iot + L) == bstar
    bincnt = (jnp.sum(jnp.where(eq0, c0, 0)) + jnp.sum(jnp.where(eq1, c1, 0)))
    return bstar, c_gt, s_d, n_d, bincnt

  def process_query(q, simr, qosr, avgr):
    def row_vd(j):
      o = pl.multiple_of(j * L, L)
      s = simr[pl.ds(o, L)]
      r = qosr[pl.ds(o, L)]
      a = avgr[pl.ds(o, L)]
      v = jnp.where(r > 0.0, s, 0.0)
      return v, r - a

    def hist_scatter(v, d, mask):
      hidx = iot * 32 + _digit(v, 0.0, 32.0)
      plsc.addupdate_scatter(histc, [hidx], ones_i, mask=mask)
      plsc.addupdate_scatter(hists, [hidx], v, mask=mask)
      plsc.addupdate_scatter(histn, [hidx], v * d, mask=mask)

    def p1(j, _):
      v, d = row_vd(j)
      hist_scatter(v, d, v > 0.0)
      return 0

    lax.fori_loop(0, NB, p1, 0, unroll=2)

    c0, c1, s0, s1, n0, n1 = fold_hists()
    count_pos = jnp.sum(c0) + jnp.sum(c1)

    def direct_path():
      return jnp.sum(s0) + jnp.sum(s1), jnp.sum(n0) + jnp.sum(n1)

    def select_path():
      bstar, c_gt, s_d, n_d, bincnt = pick_bin(c0, c1, s0, s1, n0, n1, K)

      def compact_step(v, d, m, carry):
        off, cmin_v, cmax_v = carry
        mi = m.astype(_i32)
        posn = off + plsc.cumsum(mi) - mi
        plsc.store_scatter(candv, [posn], v, mask=m)
        plsc.store_scatter(candd, [posn], d, mask=m)
        off = off + plsc.all_reduce_population_count(m)
        cmin_v = jnp.minimum(cmin_v, jnp.where(m, v, jnp.float32(3e38)))
        cmax_v = jnp.maximum(cmax_v, jnp.where(m, v, jnp.float32(-3e38)))
        return off, cmin_v, cmax_v

      def compact_init():
        return (zi, jnp.full((L,), 3e38, _f32), jnp.full((L,), -3e38, _f32))

      def rcompact(j, carry):
        v, d = row_vd(j)
        m = (v > 0.0) & (_digit(v, 0.0, 32.0) == bstar)
        return compact_step(v, d, m, carry)

      off, cmin_v, cmax_v = lax.fori_loop(0, NB, rcompact, compact_init())
      cmin = jnp.min(cmin_v)
      cmax = jnp.max(cmax_v)
      kk = K - c_gt
      ncand = bincnt
      state = jnp.where(
          ncand <= L, _i32(1),
          jnp.where((kk == ncand) | (cmin == cmax), _i32(2), _i32(0)))

      def cond(carry):
        return carry[-1] == 0

      def body(carry):
        base, scale, kk, ncand, s_ab, n_ab, rnd, _ = carry
        trips = (ncand + (L - 1)) // L

        def cand_vd(j):
          o = pl.multiple_of(j * L, L)
          return candv[pl.ds(o, L)], candd[pl.ds(o, L)], (o + iot) < ncand

        def chist(j, _):
          v, d, valid = cand_vd(j)
          hidx = iot * 32 + _digit(v, base, scale)
          plsc.addupdate_scatter(histc, [hidx], ones_i, mask=valid)
          plsc.addupdate_scatter(hists, [hidx], v, mask=valid)
          plsc.addupdate_scatter(histn, [hidx], v * d, mask=valid)
          return 0

        lax.fori_loop(0, trips, chist, 0)
        c0, c1, s0, s1, n0, n1 = fold_hists()
        bstar, c_gt, s_d, n_d, bincnt = pick_bin(c0, c1, s0, s1, n0, n1, kk)

        def ccompact(j, carry):
          v, d, valid = cand_vd(j)
          m = valid & (_digit(v, base, scale) == bstar)
          return compact_step(v, d, m, carry)

        off, cmin_v, cmax_v = lax.fori_loop(0, trips, ccompact, compact_init())
        cmin = jnp.min(cmin_v)
        cmax = jnp.max(cmax_v)
        kk2 = kk - c_gt
        ncand2 = bincnt
        rnd2 = rnd + 1
        state2 = jnp.where(
            ncand2 <= L, _i32(1),
            jnp.where((kk2 == ncand2) | (cmin == cmax) | (rnd2 >= MAXROUNDS),
                      _i32(2), _i32(0)))
        base2 = cmin
        denom = jnp.maximum(cmax - cmin, jnp.float32(1e-38))
        scale2 = (jnp.full((L,), 32.0, _f32) / jnp.full((L,), denom, _f32))[0]
        return (base2, scale2, kk2, ncand2, s_ab + s_d, n_ab + n_d, rnd2,
                state2)

      base, scale, kk, ncand, s_ab, n_ab, rnd, state = lax.while_loop(
          cond, body,
          (jnp.float32(0.0), jnp.float32(32.0), kk, ncand, s_d, n_d,
           _i32(1), state))

      def term_sort():
        v = candv[pl.ds(0, L)]
        d = candd[pl.ds(0, L)]
        valid = iot < ncand
        ve = jnp.where(valid, v, -1.0)
        de = jnp.where(valid, d, 0.0)
        sk, _ = plsc.sort_key_val(ve, de, descending=True)
        theta = jnp.sum(jnp.where(iot == kk - 1, sk, 0.0))
        cntgt = jnp.sum(jnp.where(ve > theta, 1, 0))
        mfill = kk - cntgt
        tie = ve == theta
        pref = plsc.cumsum(tie.astype(_i32))
        sel = (ve > theta) | (tie & (pref <= mfill))
        s_t = jnp.sum(jnp.where(sel, ve, 0.0))
        n_t = jnp.sum(jnp.where(sel, ve * de, 0.0))
        return s_t, n_t

      def term_first():
        trips = (kk + (L - 1)) // L

        def tf(j, carry):
          sa, na = carry
          o = pl.multiple_of(j * L, L)
          v = candv[pl.ds(o, L)]
          d = candd[pl.ds(o, L)]
          valid = (o + iot) < kk
          sa = sa + jnp.where(valid, v, 0.0)
          na = na + jnp.where(valid, v * d, 0.0)
          return sa, na

        sa, na = lax.fori_loop(0, trips, tf, (zf, zf))
        return jnp.sum(sa), jnp.sum(na)

      s_t, n_t = lax.cond(state == 1, term_sort, term_first)
      return s_ab + s_t, n_ab + n_t

    S, N = lax.cond(count_pos <= K, direct_path, select_path)
    av = plsc.load_gather(avgi, [jnp.full((L,), q, _i32)])
    ratio = (jnp.full((L,), N, _f32)
             / jnp.full((L,), S + jnp.float32(1e-8), _f32))[0]
    pred = av[0] + ratio
    plsc.store_scatter(preds, [jnp.full((L,), q, _i32)],
                       jnp.full((L,), pred, _f32), mask=iot == 0)

  def chunk(c, _):
    cs = pltpu.async_copy(sim_hbm.at[idxs.at[c]], simb, sem_s)
    cq = pltpu.async_copy(qos_hbm.at[idxq.at[c]], qosb, sem_q)
    ca = pltpu.async_copy(avg_hbm.at[idxa.at[c]], avgb, sem_a)
    cs.wait()
    cq.wait()
    ca.wait()

    def per_q(g, _):
      process_query(c * G + g, simb.at[g], qosb.at[g], avgb.at[g])
      return 0

    lax.fori_loop(0, G, per_q, 0)
    return 0

  lax.fori_loop(0, NCHUNK, chunk, 0)
  pltpu.sync_copy(preds, out_hbm.at[pl.ds(pl.multiple_of(wid * QPW, QPW),
                                          QPW)])


def kernel(qos, item_avg, item_sim_agg, user_id, item_id, time_id):
  qos2 = qos.reshape(T * U, I)
  avgflat = item_avg.reshape(T * I)
  user_id = user_id.astype(_i32)
  item_id = item_id.astype(_i32)
  time_id = time_id.astype(_i32)
  rq = (time_id * U + user_id).reshape(NW, NCHUNK, G)
  rs = item_id.reshape(NW, NCHUNK, G)
  ra = time_id.reshape(NW, NCHUNK, G)
  fia = (time_id * I + item_id).reshape(NW, QPW)

  mesh = plsc.VectorSubcoreMesh(core_axis_name="c", subcore_axis_name="s")
  qos2 = jnp.pad(qos2, ((0, 0), (0, IP - I)))
  simp = jnp.pad(item_sim_agg, ((0, 0), (0, IP - I)))
  avgp = jnp.pad(item_avg, ((0, 0), (0, IP - I)))
  f = pl.kernel(
      _sc_body,
      out_type=jax.ShapeDtypeStruct((B,), _f32),
      mesh=mesh,
      compiler_params=pltpu.CompilerParams(use_tc_tiling_on_sc=False,
                                           needs_layout_passes=False),
      scratch_types=[
          pltpu.VMEM((G, IP), _f32),
          pltpu.VMEM((G, IP), _f32),
          pltpu.VMEM((G, IP), _f32),
          pltpu.VMEM((NCHUNK, G), _i32),
          pltpu.VMEM((NCHUNK, G), _i32),
          pltpu.VMEM((NCHUNK, G), _i32),
          pltpu.VMEM((QPW,), _i32),
          pltpu.VMEM((QPW,), _f32),
          pltpu.VMEM((QPW,), _f32),
          pltpu.VMEM((CANDCAP,), _f32),
          pltpu.VMEM((CANDCAP,), _f32),
          pltpu.VMEM((512,), _i32),
          pltpu.VMEM((512,), _f32),
          pltpu.VMEM((512,), _f32),
          pltpu.SemaphoreType.DMA,
          pltpu.SemaphoreType.DMA,
          pltpu.SemaphoreType.DMA,
          pltpu.SemaphoreType.DMA,
      ],
  )
  return f(simp, qos2, avgp, avgflat, rq, rs, ra, fia)


# speedup vs baseline: 10.2401x; 1.9294x over previous
import jax
import jax.numpy as jnp
from jax import lax
from jax.experimental import pallas as pl
from jax.experimental.pallas import tpu as pltpu
from jax.experimental.pallas import tpu_sc as plsc

T, U, I = 16, 142, 4500
IP = 4608
K = 50
B = 4096
NW = 32
QPW = B // NW
G = 8
NCHUNK = QPW // G
L = 16
NB = IP // L
CANDCAP = 4608
MAXROUNDS = 24

_i32 = jnp.int32
_f32 = jnp.float32


def _iota():
  return lax.iota(_i32, L)


def _digit(v, base, scale):
  dv = jnp.clip((v - base) * scale, 0.0, 31.0)
  return dv.astype(_i32)


def _sc_body(sim_hbm, qos_hbm, avg_hbm, avgflat_hbm, rq_hbm, rs_hbm, ra_hbm,
             fia_hbm, out_hbm, simb, qosb, avgb, idxq, idxs, idxa, fi, avgi,
             preds, candv, candd, histc, hists, histn, sem_s, sem_q, sem_a,
             sem_f):
  wid = lax.axis_index("s") * 2 + lax.axis_index("c")
  iot = _iota()
  zf = jnp.zeros((L,), _f32)
  zi = jnp.zeros((L,), _i32)
  ones_i = jnp.ones((L,), _i32)

  pltpu.sync_copy(rq_hbm.at[wid], idxq)
  pltpu.sync_copy(rs_hbm.at[wid], idxs)
  pltpu.sync_copy(ra_hbm.at[wid], idxa)
  pltpu.sync_copy(fia_hbm.at[wid], fi)
  pltpu.async_copy(avgflat_hbm.at[fi], avgi, sem_f).wait()

  for h, z in ((histc, zi), (hists, zf), (histn, zf)):
    for j in range(512 // L):
      h[pl.ds(j * L, L)] = z

  def fold_hists():
    c0 = zi
    c1 = zi
    s0 = zf
    s1 = zf
    n0 = zf
    n1 = zf
    for lane in range(L):
      o = lane * 32
      c0 = c0 + histc[pl.ds(o, L)]
      c1 = c1 + histc[pl.ds(o + L, L)]
      s0 = s0 + hists[pl.ds(o, L)]
      s1 = s1 + hists[pl.ds(o + L, L)]
      n0 = n0 + histn[pl.ds(o, L)]
      n1 = n1 + histn[pl.ds(o + L, L)]
      histc[pl.ds(o, L)] = zi
      histc[pl.ds(o + L, L)] = zi
      hists[pl.ds(o, L)] = zf
      hists[pl.ds(o + L, L)] = zf
      histn[pl.ds(o, L)] = zf
      histn[pl.ds(o + L, L)] = zf
    return c0, c1, s0, s1, n0, n1

  def pick_bin(c0, c1, s0, s1, n0, n1, kk):
    t1 = jnp.sum(c1)
    rc0 = lax.rev(plsc.cumsum(lax.rev(c0, (0,))), (0,)) + t1
    rc1 = lax.rev(plsc.cumsum(lax.rev(c1, (0,))), (0,))
    b0 = jnp.max(jnp.where(rc0 >= kk, iot, -1))
    b1 = jnp.max(jnp.where(rc1 >= kk, iot + L, -1))
    bstar = jnp.maximum(b0, b1)
    gt0 = iot > bstar
    gt1 = (iot + L) > bstar
    c_gt = jnp.sum(jnp.where(gt0, c0, 0)) + jnp.sum(jnp.where(gt1, c1, 0))
    s_d = jnp.sum(jnp.where(gt0, s0, 0.0)) + jnp.sum(jnp.where(gt1, s1, 0.0))
    n_d = jnp.sum(jnp.where(gt0, n0, 0.0)) + jnp.sum(jnp.where(gt1, n1, 0.0))
    eq0 = iot == bstar
    eq1 = (iot + L) == bstar
    bincnt = (jnp.sum(jnp.where(eq0, c0, 0)) + jnp.sum(jnp.where(eq1, c1, 0)))
    return bstar, c_gt, s_d, n_d, bincnt

  def process_query(q, simr, qosr, avgr):
    def row_vd(j):
      o = pl.multiple_of(j * L, L)
      s = simr[pl.ds(o, L)]
      r = qosr[pl.ds(o, L)]
      a = avgr[pl.ds(o, L)]
      v = jnp.where(r > 0.0, s, 0.0)
      return v, r - a

    def hist_scatter(v, d, mask):
      hidx = iot * 32 + _digit(v, 0.0, 32.0)
      plsc.addupdate_scatter(histc, [hidx], ones_i, mask=mask)
      plsc.addupdate_scatter(hists, [hidx], v, mask=mask)
      plsc.addupdate_scatter(histn, [hidx], v * d, mask=mask)

    @plsc.parallel_loop(0, NB, unroll=8)
    def p1(j):
      v, d = row_vd(j)
      hist_scatter(v, d, v > 0.0)

    c0, c1, s0, s1, n0, n1 = fold_hists()
    count_pos = jnp.sum(c0) + jnp.sum(c1)

    def direct_path():
      return jnp.sum(s0) + jnp.sum(s1), jnp.sum(n0) + jnp.sum(n1)

    def select_path():
      bstar, c_gt, s_d, n_d, bincnt = pick_bin(c0, c1, s0, s1, n0, n1, K)

      def compact_step(v, d, m, carry):
        off, cmin_v, cmax_v = carry
        mi = m.astype(_i32)
        posn = off + plsc.cumsum(mi) - mi
        plsc.store_scatter(candv, [posn], v, mask=m)
        plsc.store_scatter(candd, [posn], d, mask=m)
        off = off + plsc.all_reduce_population_count(m)
        cmin_v = jnp.minimum(cmin_v, jnp.where(m, v, jnp.float32(3e38)))
        cmax_v = jnp.maximum(cmax_v, jnp.where(m, v, jnp.float32(-3e38)))
        return off, cmin_v, cmax_v

      def compact_init():
        return (zi, jnp.full((L,), 3e38, _f32), jnp.full((L,), -3e38, _f32))

      @plsc.parallel_loop(0, NB, unroll=4, carry=compact_init())
      def rcompact(j, carry):
        v, d = row_vd(j)
        m = (v > 0.0) & (_digit(v, 0.0, 32.0) == bstar)
        return compact_step(v, d, m, carry)

      off, cmin_v, cmax_v = rcompact
      cmin = jnp.min(cmin_v)
      cmax = jnp.max(cmax_v)
      kk = K - c_gt
      ncand = bincnt
      state = jnp.where(
          ncand <= L, _i32(1),
          jnp.where((kk == ncand) | (cmin == cmax), _i32(2), _i32(0)))

      def cond(carry):
        return carry[-1] == 0

      def body(carry):
        base, scale, kk, ncand, s_ab, n_ab, rnd, _ = carry
        trips = (ncand + (L - 1)) // L

        def cand_vd(j):
          o = pl.multiple_of(j * L, L)
          return candv[pl.ds(o, L)], candd[pl.ds(o, L)], (o + iot) < ncand

        @plsc.parallel_loop(0, trips, unroll=2)
        def chist(j):
          v, d, valid = cand_vd(j)
          hidx = iot * 32 + _digit(v, base, scale)
          plsc.addupdate_scatter(histc, [hidx], ones_i, mask=valid)
          plsc.addupdate_scatter(hists, [hidx], v, mask=valid)
          plsc.addupdate_scatter(histn, [hidx], v * d, mask=valid)
        c0, c1, s0, s1, n0, n1 = fold_hists()
        bstar, c_gt, s_d, n_d, bincnt = pick_bin(c0, c1, s0, s1, n0, n1, kk)

        @plsc.parallel_loop(0, trips, unroll=2, carry=compact_init())
        def ccompact(j, carry):
          v, d, valid = cand_vd(j)
          m = valid & (_digit(v, base, scale) == bstar)
          return compact_step(v, d, m, carry)

        off, cmin_v, cmax_v = ccompact
        cmin = jnp.min(cmin_v)
        cmax = jnp.max(cmax_v)
        kk2 = kk - c_gt
        ncand2 = bincnt
        rnd2 = rnd + 1
        state2 = jnp.where(
            ncand2 <= L, _i32(1),
            jnp.where((kk2 == ncand2) | (cmin == cmax) | (rnd2 >= MAXROUNDS),
                      _i32(2), _i32(0)))
        base2 = cmin
        denom = jnp.maximum(cmax - cmin, jnp.float32(1e-38))
        scale2 = (jnp.full((L,), 32.0, _f32) / jnp.full((L,), denom, _f32))[0]
        return (base2, scale2, kk2, ncand2, s_ab + s_d, n_ab + n_d, rnd2,
                state2)

      base, scale, kk, ncand, s_ab, n_ab, rnd, state = lax.while_loop(
          cond, body,
          (jnp.float32(0.0), jnp.float32(32.0), kk, ncand, s_d, n_d,
           _i32(1), state))

      def term_sort():
        v = candv[pl.ds(0, L)]
        d = candd[pl.ds(0, L)]
        valid = iot < ncand
        ve = jnp.where(valid, v, -1.0)
        de = jnp.where(valid, d, 0.0)
        sk, _ = plsc.sort_key_val(ve, de, descending=True)
        theta = jnp.sum(jnp.where(iot == kk - 1, sk, 0.0))
        cntgt = jnp.sum(jnp.where(ve > theta, 1, 0))
        mfill = kk - cntgt
        tie = ve == theta
        pref = plsc.cumsum(tie.astype(_i32))
        sel = (ve > theta) | (tie & (pref <= mfill))
        s_t = jnp.sum(jnp.where(sel, ve, 0.0))
        n_t = jnp.sum(jnp.where(sel, ve * de, 0.0))
        return s_t, n_t

      def term_first():
        trips = (kk + (L - 1)) // L

        def tf(j, carry):
          sa, na = carry
          o = pl.multiple_of(j * L, L)
          v = candv[pl.ds(o, L)]
          d = candd[pl.ds(o, L)]
          valid = (o + iot) < kk
          sa = sa + jnp.where(valid, v, 0.0)
          na = na + jnp.where(valid, v * d, 0.0)
          return sa, na

        sa, na = lax.fori_loop(0, trips, tf, (zf, zf))
        return jnp.sum(sa), jnp.sum(na)

      s_t, n_t = lax.cond(state == 1, term_sort, term_first)
      return s_ab + s_t, n_ab + n_t

    S, N = lax.cond(count_pos <= K, direct_path, select_path)
    av = plsc.load_gather(avgi, [jnp.full((L,), q, _i32)])
    ratio = (jnp.full((L,), N, _f32)
             / jnp.full((L,), S + jnp.float32(1e-8), _f32))[0]
    pred = av[0] + ratio
    plsc.store_scatter(preds, [jnp.full((L,), q, _i32)],
                       jnp.full((L,), pred, _f32), mask=iot == 0)

  def chunk(c, _):
    cs = pltpu.async_copy(sim_hbm.at[idxs.at[c]], simb, sem_s)
    cq = pltpu.async_copy(qos_hbm.at[idxq.at[c]], qosb, sem_q)
    ca = pltpu.async_copy(avg_hbm.at[idxa.at[c]], avgb, sem_a)
    cs.wait()
    cq.wait()
    ca.wait()

    def per_q(g, _):
      process_query(c * G + g, simb.at[g], qosb.at[g], avgb.at[g])
      return 0

    lax.fori_loop(0, G, per_q, 0)
    return 0

  lax.fori_loop(0, NCHUNK, chunk, 0)
  pltpu.sync_copy(preds, out_hbm.at[pl.ds(pl.multiple_of(wid * QPW, QPW),
                                          QPW)])


def kernel(qos, item_avg, item_sim_agg, user_id, item_id, time_id):
  qos2 = qos.reshape(T * U, I)
  avgflat = item_avg.reshape(T * I)
  user_id = user_id.astype(_i32)
  item_id = item_id.astype(_i32)
  time_id = time_id.astype(_i32)
  rq = (time_id * U + user_id).reshape(NW, NCHUNK, G)
  rs = item_id.reshape(NW, NCHUNK, G)
  ra = time_id.reshape(NW, NCHUNK, G)
  fia = (time_id * I + item_id).reshape(NW, QPW)

  mesh = plsc.VectorSubcoreMesh(core_axis_name="c", subcore_axis_name="s")
  qos2 = jnp.pad(qos2, ((0, 0), (0, IP - I)))
  simp = jnp.pad(item_sim_agg, ((0, 0), (0, IP - I)))
  avgp = jnp.pad(item_avg, ((0, 0), (0, IP - I)))
  f = pl.kernel(
      _sc_body,
      out_type=jax.ShapeDtypeStruct((B,), _f32),
      mesh=mesh,
      compiler_params=pltpu.CompilerParams(use_tc_tiling_on_sc=False,
                                           needs_layout_passes=False),
      scratch_types=[
          pltpu.VMEM((G, IP), _f32),
          pltpu.VMEM((G, IP), _f32),
          pltpu.VMEM((G, IP), _f32),
          pltpu.VMEM((NCHUNK, G), _i32),
          pltpu.VMEM((NCHUNK, G), _i32),
          pltpu.VMEM((NCHUNK, G), _i32),
          pltpu.VMEM((QPW,), _i32),
          pltpu.VMEM((QPW,), _f32),
          pltpu.VMEM((QPW,), _f32),
          pltpu.VMEM((CANDCAP,), _f32),
          pltpu.VMEM((CANDCAP,), _f32),
          pltpu.VMEM((512,), _i32),
          pltpu.VMEM((512,), _f32),
          pltpu.VMEM((512,), _f32),
          pltpu.SemaphoreType.DMA,
          pltpu.SemaphoreType.DMA,
          pltpu.SemaphoreType.DMA,
          pltpu.SemaphoreType.DMA,
      ],
  )
  return f(simp, qos2, avgp, avgflat, rq, rs, ra, fia)


# speedup vs baseline: 13.0061x; 1.2701x over previous
import jax
import jax.numpy as jnp
from jax import lax
from jax.experimental import pallas as pl
from jax.experimental.pallas import tpu as pltpu
from jax.experimental.pallas import tpu_sc as plsc

T, U, I = 16, 142, 4500
IP = 4608
K = 50
B = 4096
NW = 32
QPW = B // NW
G = 8
NCHUNK = QPW // G
L = 16
NB = IP // L
CANDCAP = 4608
MAXROUNDS = 24

_i32 = jnp.int32
_f32 = jnp.float32


def _iota():
  return lax.iota(_i32, L)


def _digit(v, base, scale):
  dv = jnp.clip((v - base) * scale, 0.0, 31.0)
  return dv.astype(_i32)


def _sc_body(sim_hbm, qos_hbm, avg_hbm, avgflat_hbm, rq_hbm, rs_hbm, ra_hbm,
             fia_hbm, out_hbm, simb, qosb, avgb, idxq, idxs, idxa, fi, avgi,
             preds, candv, candd, histc, hists, histn, sem_s, sem_q, sem_a,
             sem_f):
  wid = lax.axis_index("s") * 2 + lax.axis_index("c")
  iot = _iota()
  zf = jnp.zeros((L,), _f32)
  zi = jnp.zeros((L,), _i32)
  ones_i = jnp.ones((L,), _i32)

  pltpu.sync_copy(rq_hbm.at[wid], idxq)
  pltpu.sync_copy(rs_hbm.at[wid], idxs)
  pltpu.sync_copy(ra_hbm.at[wid], idxa)
  pltpu.sync_copy(fia_hbm.at[wid], fi)
  pltpu.async_copy(avgflat_hbm.at[fi], avgi, sem_f).wait()

  for h, z in ((histc, zi), (hists, zf), (histn, zf)):
    for j in range(512 // L):
      h[pl.ds(j * L, L)] = z

  def fold_hists():
    c0 = zi
    c1 = zi
    s0 = zf
    s1 = zf
    n0 = zf
    n1 = zf
    for lane in range(L):
      o = lane * 32
      idx0 = o + ((iot + lane) & 31)
      idx1 = o + ((iot + L + lane) & 31)
      c0 = c0 + plsc.load_gather(histc, [idx0])
      c1 = c1 + plsc.load_gather(histc, [idx1])
      s0 = s0 + plsc.load_gather(hists, [idx0])
      s1 = s1 + plsc.load_gather(hists, [idx1])
      n0 = n0 + plsc.load_gather(histn, [idx0])
      n1 = n1 + plsc.load_gather(histn, [idx1])
      plsc.store_scatter(histc, [idx0], zi)
      plsc.store_scatter(histc, [idx1], zi)
      plsc.store_scatter(hists, [idx0], zf)
      plsc.store_scatter(hists, [idx1], zf)
      plsc.store_scatter(histn, [idx0], zf)
      plsc.store_scatter(histn, [idx1], zf)
    return c0, c1, s0, s1, n0, n1

  def pick_bin(c0, c1, s0, s1, n0, n1, kk):
    t1 = jnp.sum(c1)
    rc0 = lax.rev(plsc.cumsum(lax.rev(c0, (0,))), (0,)) + t1
    rc1 = lax.rev(plsc.cumsum(lax.rev(c1, (0,))), (0,))
    b0 = jnp.max(jnp.where(rc0 >= kk, iot, -1))
    b1 = jnp.max(jnp.where(rc1 >= kk, iot + L, -1))
    bstar = jnp.maximum(b0, b1)
    gt0 = iot > bstar
    gt1 = (iot + L) > bstar
    c_gt = jnp.sum(jnp.where(gt0, c0, 0)) + jnp.sum(jnp.where(gt1, c1, 0))
    s_d = jnp.sum(jnp.where(gt0, s0, 0.0)) + jnp.sum(jnp.where(gt1, s1, 0.0))
    n_d = jnp.sum(jnp.where(gt0, n0, 0.0)) + jnp.sum(jnp.where(gt1, n1, 0.0))
    eq0 = iot == bstar
    eq1 = (iot + L) == bstar
    bincnt = (jnp.sum(jnp.where(eq0, c0, 0)) + jnp.sum(jnp.where(eq1, c1, 0)))
    return bstar, c_gt, s_d, n_d, bincnt

  def process_query(q, simr, qosr, avgr):
    def row_vd(j):
      o = pl.multiple_of(j * L, L)
      s = simr[pl.ds(o, L)]
      r = qosr[pl.ds(o, L)]
      a = avgr[pl.ds(o, L)]
      v = jnp.where(r > 0.0, s, 0.0)
      return v, r - a

    def hist_scatter(v, d, mask):
      dig = _digit(v, 0.0, 32.0)
      hidx = iot * 32 + ((dig + iot) & 31)
      plsc.addupdate_scatter(histc, [hidx], ones_i, mask=mask)
      plsc.addupdate_scatter(hists, [hidx], v, mask=mask)
      plsc.addupdate_scatter(histn, [hidx], v * d, mask=mask)

    @plsc.parallel_loop(0, NB, unroll=8)
    def p1(j):
      v, d = row_vd(j)
      hist_scatter(v, d, v > 0.0)

    c0, c1, s0, s1, n0, n1 = fold_hists()
    count_pos = jnp.sum(c0) + jnp.sum(c1)

    def direct_path():
      return jnp.sum(s0) + jnp.sum(s1), jnp.sum(n0) + jnp.sum(n1)

    def select_path():
      bstar, c_gt, s_d, n_d, bincnt = pick_bin(c0, c1, s0, s1, n0, n1, K)

      def compact_step(v, d, m, carry):
        off, cmin_v, cmax_v = carry
        mi = m.astype(_i32)
        posn = off + plsc.cumsum(mi) - mi
        plsc.store_scatter(candv, [posn], v, mask=m)
        plsc.store_scatter(candd, [posn], d, mask=m)
        off = off + plsc.all_reduce_population_count(m)
        cmin_v = jnp.minimum(cmin_v, jnp.where(m, v, jnp.float32(3e38)))
        cmax_v = jnp.maximum(cmax_v, jnp.where(m, v, jnp.float32(-3e38)))
        return off, cmin_v, cmax_v

      def compact_init():
        return (zi, jnp.full((L,), 3e38, _f32), jnp.full((L,), -3e38, _f32))

      @plsc.parallel_loop(0, NB, unroll=4, carry=compact_init())
      def rcompact(j, carry):
        v, d = row_vd(j)
        m = (v > 0.0) & (_digit(v, 0.0, 32.0) == bstar)
        return compact_step(v, d, m, carry)

      off, cmin_v, cmax_v = rcompact
      cmin = jnp.min(cmin_v)
      cmax = jnp.max(cmax_v)
      kk = K - c_gt
      ncand = bincnt
      state = jnp.where(
          ncand <= L, _i32(1),
          jnp.where((kk == ncand) | (cmin == cmax), _i32(2), _i32(0)))

      def cond(carry):
        return carry[-1] == 0

      def body(carry):
        base, scale, kk, ncand, s_ab, n_ab, rnd, _ = carry
        trips = (ncand + (L - 1)) // L

        def cand_vd(j):
          o = pl.multiple_of(j * L, L)
          return candv[pl.ds(o, L)], candd[pl.ds(o, L)], (o + iot) < ncand

        @plsc.parallel_loop(0, trips, unroll=2)
        def chist(j):
          v, d, valid = cand_vd(j)
          hidx = iot * 32 + ((_digit(v, base, scale) + iot) & 31)
          plsc.addupdate_scatter(histc, [hidx], ones_i, mask=valid)
          plsc.addupdate_scatter(hists, [hidx], v, mask=valid)
          plsc.addupdate_scatter(histn, [hidx], v * d, mask=valid)
        c0, c1, s0, s1, n0, n1 = fold_hists()
        bstar, c_gt, s_d, n_d, bincnt = pick_bin(c0, c1, s0, s1, n0, n1, kk)

        @plsc.parallel_loop(0, trips, unroll=2, carry=compact_init())
        def ccompact(j, carry):
          v, d, valid = cand_vd(j)
          m = valid & (_digit(v, base, scale) == bstar)
          return compact_step(v, d, m, carry)

        off, cmin_v, cmax_v = ccompact
        cmin = jnp.min(cmin_v)
        cmax = jnp.max(cmax_v)
        kk2 = kk - c_gt
        ncand2 = bincnt
        rnd2 = rnd + 1
        state2 = jnp.where(
            ncand2 <= L, _i32(1),
            jnp.where((kk2 == ncand2) | (cmin == cmax) | (rnd2 >= MAXROUNDS),
                      _i32(2), _i32(0)))
        base2 = cmin
        denom = jnp.maximum(cmax - cmin, jnp.float32(1e-38))
        scale2 = (jnp.full((L,), 32.0, _f32) / jnp.full((L,), denom, _f32))[0]
        return (base2, scale2, kk2, ncand2, s_ab + s_d, n_ab + n_d, rnd2,
                state2)

      base, scale, kk, ncand, s_ab, n_ab, rnd, state = lax.while_loop(
          cond, body,
          (jnp.float32(0.0), jnp.float32(32.0), kk, ncand, s_d, n_d,
           _i32(1), state))

      def term_sort():
        v = candv[pl.ds(0, L)]
        d = candd[pl.ds(0, L)]
        valid = iot < ncand
        ve = jnp.where(valid, v, -1.0)
        de = jnp.where(valid, d, 0.0)
        sk, _ = plsc.sort_key_val(ve, de, descending=True)
        theta = jnp.sum(jnp.where(iot == kk - 1, sk, 0.0))
        cntgt = jnp.sum(jnp.where(ve > theta, 1, 0))
        mfill = kk - cntgt
        tie = ve == theta
        pref = plsc.cumsum(tie.astype(_i32))
        sel = (ve > theta) | (tie & (pref <= mfill))
        s_t = jnp.sum(jnp.where(sel, ve, 0.0))
        n_t = jnp.sum(jnp.where(sel, ve * de, 0.0))
        return s_t, n_t

      def term_first():
        trips = (kk + (L - 1)) // L

        def tf(j, carry):
          sa, na = carry
          o = pl.multiple_of(j * L, L)
          v = candv[pl.ds(o, L)]
          d = candd[pl.ds(o, L)]
          valid = (o + iot) < kk
          sa = sa + jnp.where(valid, v, 0.0)
          na = na + jnp.where(valid, v * d, 0.0)
          return sa, na

        sa, na = lax.fori_loop(0, trips, tf, (zf, zf))
        return jnp.sum(sa), jnp.sum(na)

      s_t, n_t = lax.cond(state == 1, term_sort, term_first)
      return s_ab + s_t, n_ab + n_t

    S, N = lax.cond(count_pos <= K, direct_path, select_path)
    av = plsc.load_gather(avgi, [jnp.full((L,), q, _i32)])
    ratio = (jnp.full((L,), N, _f32)
             / jnp.full((L,), S + jnp.float32(1e-8), _f32))[0]
    pred = av[0] + ratio
    plsc.store_scatter(preds, [jnp.full((L,), q, _i32)],
                       jnp.full((L,), pred, _f32), mask=iot == 0)

  def chunk(c, _):
    cs = pltpu.async_copy(sim_hbm.at[idxs.at[c]], simb, sem_s)
    cq = pltpu.async_copy(qos_hbm.at[idxq.at[c]], qosb, sem_q)
    ca = pltpu.async_copy(avg_hbm.at[idxa.at[c]], avgb, sem_a)
    cs.wait()
    cq.wait()
    ca.wait()

    def per_q(g, _):
      process_query(c * G + g, simb.at[g], qosb.at[g], avgb.at[g])
      return 0

    lax.fori_loop(0, G, per_q, 0)
    return 0

  lax.fori_loop(0, NCHUNK, chunk, 0)
  pltpu.sync_copy(preds, out_hbm.at[pl.ds(pl.multiple_of(wid * QPW, QPW),
                                          QPW)])


def kernel(qos, item_avg, item_sim_agg, user_id, item_id, time_id):
  qos2 = qos.reshape(T * U, I)
  avgflat = item_avg.reshape(T * I)
  user_id = user_id.astype(_i32)
  item_id = item_id.astype(_i32)
  time_id = time_id.astype(_i32)
  rq = (time_id * U + user_id).reshape(NW, NCHUNK, G)
  rs = item_id.reshape(NW, NCHUNK, G)
  ra = time_id.reshape(NW, NCHUNK, G)
  fia = (time_id * I + item_id).reshape(NW, QPW)

  mesh = plsc.VectorSubcoreMesh(core_axis_name="c", subcore_axis_name="s")
  qos2 = jnp.pad(qos2, ((0, 0), (0, IP - I)))
  simp = jnp.pad(item_sim_agg, ((0, 0), (0, IP - I)))
  avgp = jnp.pad(item_avg, ((0, 0), (0, IP - I)))
  f = pl.kernel(
      _sc_body,
      out_type=jax.ShapeDtypeStruct((B,), _f32),
      mesh=mesh,
      compiler_params=pltpu.CompilerParams(use_tc_tiling_on_sc=False,
                                           needs_layout_passes=False),
      scratch_types=[
          pltpu.VMEM((G, IP), _f32),
          pltpu.VMEM((G, IP), _f32),
          pltpu.VMEM((G, IP), _f32),
          pltpu.VMEM((NCHUNK, G), _i32),
          pltpu.VMEM((NCHUNK, G), _i32),
          pltpu.VMEM((NCHUNK, G), _i32),
          pltpu.VMEM((QPW,), _i32),
          pltpu.VMEM((QPW,), _f32),
          pltpu.VMEM((QPW,), _f32),
          pltpu.VMEM((CANDCAP,), _f32),
          pltpu.VMEM((CANDCAP,), _f32),
          pltpu.VMEM((512,), _i32),
          pltpu.VMEM((512,), _f32),
          pltpu.VMEM((512,), _f32),
          pltpu.SemaphoreType.DMA,
          pltpu.SemaphoreType.DMA,
          pltpu.SemaphoreType.DMA,
          pltpu.SemaphoreType.DMA,
      ],
  )
  return f(simp, qos2, avgp, avgflat, rq, rs, ra, fia)


# speedup vs baseline: 13.4626x; 1.0351x over previous
import jax
import jax.numpy as jnp
from jax import lax
from jax.experimental import pallas as pl
from jax.experimental.pallas import tpu as pltpu
from jax.experimental.pallas import tpu_sc as plsc

T, U, I = 16, 142, 4500
IP = 4608
K = 50
B = 4096
NW = 32
QPW = B // NW
G = 8
NCHUNK = QPW // G
L = 16
NB = IP // L
CANDCAP = 4608
MAXROUNDS = 24

_i32 = jnp.int32
_f32 = jnp.float32


def _iota():
  return lax.iota(_i32, L)


def _digit(v, base, scale):
  dv = jnp.clip((v - base) * scale, 0.0, 31.0)
  return dv.astype(_i32)


def _sc_body(sim_hbm, qd_hbm, avgflat_hbm, rq_hbm, rs_hbm,
             fia_hbm, out_hbm, simb, qdb, idxq, idxs, fi, avgi,
             preds, candv, candd, histc, hists, histn, sem_s, sem_q,
             sem_f):
  wid = lax.axis_index("s") * 2 + lax.axis_index("c")
  iot = _iota()
  zf = jnp.zeros((L,), _f32)
  zi = jnp.zeros((L,), _i32)
  ones_i = jnp.ones((L,), _i32)

  pltpu.sync_copy(rq_hbm.at[wid], idxq)
  pltpu.sync_copy(rs_hbm.at[wid], idxs)
  pltpu.sync_copy(fia_hbm.at[wid], fi)
  pltpu.async_copy(avgflat_hbm.at[fi], avgi, sem_f).wait()

  for h, z in ((histc, zi), (hists, zf), (histn, zf)):
    for j in range(512 // L):
      h[pl.ds(j * L, L)] = z

  def fold_hists():
    c0 = zi
    c1 = zi
    s0 = zf
    s1 = zf
    n0 = zf
    n1 = zf
    for lane in range(L):
      o = lane * 32
      idx0 = o + ((iot + lane) & 31)
      idx1 = o + ((iot + L + lane) & 31)
      c0 = c0 + plsc.load_gather(histc, [idx0])
      c1 = c1 + plsc.load_gather(histc, [idx1])
      s0 = s0 + plsc.load_gather(hists, [idx0])
      s1 = s1 + plsc.load_gather(hists, [idx1])
      n0 = n0 + plsc.load_gather(histn, [idx0])
      n1 = n1 + plsc.load_gather(histn, [idx1])
      plsc.store_scatter(histc, [idx0], zi)
      plsc.store_scatter(histc, [idx1], zi)
      plsc.store_scatter(hists, [idx0], zf)
      plsc.store_scatter(hists, [idx1], zf)
      plsc.store_scatter(histn, [idx0], zf)
      plsc.store_scatter(histn, [idx1], zf)
    return c0, c1, s0, s1, n0, n1

  def pick_bin(c0, c1, s0, s1, n0, n1, kk):
    t1 = jnp.sum(c1)
    rc0 = lax.rev(plsc.cumsum(lax.rev(c0, (0,))), (0,)) + t1
    rc1 = lax.rev(plsc.cumsum(lax.rev(c1, (0,))), (0,))
    b0 = jnp.max(jnp.where(rc0 >= kk, iot, -1))
    b1 = jnp.max(jnp.where(rc1 >= kk, iot + L, -1))
    bstar = jnp.maximum(b0, b1)
    gt0 = iot > bstar
    gt1 = (iot + L) > bstar
    c_gt = jnp.sum(jnp.where(gt0, c0, 0)) + jnp.sum(jnp.where(gt1, c1, 0))
    s_d = jnp.sum(jnp.where(gt0, s0, 0.0)) + jnp.sum(jnp.where(gt1, s1, 0.0))
    n_d = jnp.sum(jnp.where(gt0, n0, 0.0)) + jnp.sum(jnp.where(gt1, n1, 0.0))
    eq0 = iot == bstar
    eq1 = (iot + L) == bstar
    bincnt = (jnp.sum(jnp.where(eq0, c0, 0)) + jnp.sum(jnp.where(eq1, c1, 0)))
    return bstar, c_gt, s_d, n_d, bincnt

  def process_query(q, simr, qdr):
    def row_vd(j):
      o = pl.multiple_of(j * L, L)
      s = simr[pl.ds(o, L)]
      d = qdr[pl.ds(o, L)]
      v = jnp.where(d == d, s, 0.0)
      return v, d

    def hist_scatter(v, d, mask):
      dig = _digit(v, 0.0, 32.0)
      hidx = iot * 32 + ((dig + iot) & 31)
      plsc.addupdate_scatter(histc, [hidx], ones_i, mask=mask)
      plsc.addupdate_scatter(hists, [hidx], v, mask=mask)
      plsc.addupdate_scatter(histn, [hidx], v * d, mask=mask)

    @plsc.parallel_loop(0, NB, unroll=8)
    def p1(j):
      v, d = row_vd(j)
      hist_scatter(v, d, v > 0.0)

    c0, c1, s0, s1, n0, n1 = fold_hists()
    count_pos = jnp.sum(c0) + jnp.sum(c1)

    def direct_path():
      return jnp.sum(s0) + jnp.sum(s1), jnp.sum(n0) + jnp.sum(n1)

    def select_path():
      bstar, c_gt, s_d, n_d, bincnt = pick_bin(c0, c1, s0, s1, n0, n1, K)

      def compact_step(v, d, m, carry):
        off, cmin_v, cmax_v = carry
        mi = m.astype(_i32)
        posn = off + plsc.cumsum(mi) - mi
        plsc.store_scatter(candv, [posn], v, mask=m)
        plsc.store_scatter(candd, [posn], d, mask=m)
        off = off + plsc.all_reduce_population_count(m)
        cmin_v = jnp.minimum(cmin_v, jnp.where(m, v, jnp.float32(3e38)))
        cmax_v = jnp.maximum(cmax_v, jnp.where(m, v, jnp.float32(-3e38)))
        return off, cmin_v, cmax_v

      def compact_init():
        return (zi, jnp.full((L,), 3e38, _f32), jnp.full((L,), -3e38, _f32))

      @plsc.parallel_loop(0, NB, unroll=4, carry=compact_init())
      def rcompact(j, carry):
        v, d = row_vd(j)
        m = (v > 0.0) & (_digit(v, 0.0, 32.0) == bstar)
        return compact_step(v, d, m, carry)

      off, cmin_v, cmax_v = rcompact
      cmin = jnp.min(cmin_v)
      cmax = jnp.max(cmax_v)
      kk = K - c_gt
      ncand = bincnt
      state = jnp.where(
          ncand <= L, _i32(1),
          jnp.where((kk == ncand) | (cmin == cmax), _i32(2), _i32(0)))

      def cond(carry):
        return carry[-1] == 0

      def body(carry):
        base, scale, kk, ncand, s_ab, n_ab, rnd, _ = carry
        trips = (ncand + (L - 1)) // L

        def cand_vd(j):
          o = pl.multiple_of(j * L, L)
          return candv[pl.ds(o, L)], candd[pl.ds(o, L)], (o + iot) < ncand

        @plsc.parallel_loop(0, trips, unroll=2)
        def chist(j):
          v, d, valid = cand_vd(j)
          hidx = iot * 32 + ((_digit(v, base, scale) + iot) & 31)
          plsc.addupdate_scatter(histc, [hidx], ones_i, mask=valid)
          plsc.addupdate_scatter(hists, [hidx], v, mask=valid)
          plsc.addupdate_scatter(histn, [hidx], v * d, mask=valid)
        c0, c1, s0, s1, n0, n1 = fold_hists()
        bstar, c_gt, s_d, n_d, bincnt = pick_bin(c0, c1, s0, s1, n0, n1, kk)

        @plsc.parallel_loop(0, trips, unroll=2, carry=compact_init())
        def ccompact(j, carry):
          v, d, valid = cand_vd(j)
          m = valid & (_digit(v, base, scale) == bstar)
          return compact_step(v, d, m, carry)

        off, cmin_v, cmax_v = ccompact
        cmin = jnp.min(cmin_v)
        cmax = jnp.max(cmax_v)
        kk2 = kk - c_gt
        ncand2 = bincnt
        rnd2 = rnd + 1
        state2 = jnp.where(
            ncand2 <= L, _i32(1),
            jnp.where((kk2 == ncand2) | (cmin == cmax) | (rnd2 >= MAXROUNDS),
                      _i32(2), _i32(0)))
        base2 = cmin
        denom = jnp.maximum(cmax - cmin, jnp.float32(1e-38))
        scale2 = (jnp.full((L,), 32.0, _f32) / jnp.full((L,), denom, _f32))[0]
        return (base2, scale2, kk2, ncand2, s_ab + s_d, n_ab + n_d, rnd2,
                state2)

      base, scale, kk, ncand, s_ab, n_ab, rnd, state = lax.while_loop(
          cond, body,
          (jnp.float32(0.0), jnp.float32(32.0), kk, ncand, s_d, n_d,
           _i32(1), state))

      def term_sort():
        v = candv[pl.ds(0, L)]
        d = candd[pl.ds(0, L)]
        valid = iot < ncand
        ve = jnp.where(valid, v, -1.0)
        de = jnp.where(valid, d, 0.0)
        sk, _ = plsc.sort_key_val(ve, de, descending=True)
        theta = jnp.sum(jnp.where(iot == kk - 1, sk, 0.0))
        cntgt = jnp.sum(jnp.where(ve > theta, 1, 0))
        mfill = kk - cntgt
        tie = ve == theta
        pref = plsc.cumsum(tie.astype(_i32))
        sel = (ve > theta) | (tie & (pref <= mfill))
        s_t = jnp.sum(jnp.where(sel, ve, 0.0))
        n_t = jnp.sum(jnp.where(sel, ve * de, 0.0))
        return s_t, n_t

      def term_first():
        trips = (kk + (L - 1)) // L

        def tf(j, carry):
          sa, na = carry
          o = pl.multiple_of(j * L, L)
          v = candv[pl.ds(o, L)]
          d = candd[pl.ds(o, L)]
          valid = (o + iot) < kk
          sa = sa + jnp.where(valid, v, 0.0)
          na = na + jnp.where(valid, v * d, 0.0)
          return sa, na

        sa, na = lax.fori_loop(0, trips, tf, (zf, zf))
        return jnp.sum(sa), jnp.sum(na)

      s_t, n_t = lax.cond(state == 1, term_sort, term_first)
      return s_ab + s_t, n_ab + n_t

    S, N = lax.cond(count_pos <= K, direct_path, select_path)
    av = plsc.load_gather(avgi, [jnp.full((L,), q, _i32)])
    ratio = (jnp.full((L,), N, _f32)
             / jnp.full((L,), S + jnp.float32(1e-8), _f32))[0]
    pred = av[0] + ratio
    plsc.store_scatter(preds, [jnp.full((L,), q, _i32)],
                       jnp.full((L,), pred, _f32), mask=iot == 0)

  def chunk(c, _):
    cs = pltpu.async_copy(sim_hbm.at[idxs.at[c]], simb, sem_s)
    cq = pltpu.async_copy(qd_hbm.at[idxq.at[c]], qdb, sem_q)
    cs.wait()
    cq.wait()

    def per_q(g, _):
      process_query(c * G + g, simb.at[g], qdb.at[g])
      return 0

    lax.fori_loop(0, G, per_q, 0)
    return 0

  lax.fori_loop(0, NCHUNK, chunk, 0)
  pltpu.sync_copy(preds, out_hbm.at[pl.ds(pl.multiple_of(wid * QPW, QPW),
                                          QPW)])


def kernel(qos, item_avg, item_sim_agg, user_id, item_id, time_id):
  qd = jnp.where(qos > 0.0, qos - item_avg[:, None, :],
                 jnp.float32(jnp.nan)).reshape(T * U, I)
  avgflat = item_avg.reshape(T * I)
  user_id = user_id.astype(_i32)
  item_id = item_id.astype(_i32)
  time_id = time_id.astype(_i32)
  rq = (time_id * U + user_id).reshape(NW, NCHUNK, G)
  rs = item_id.reshape(NW, NCHUNK, G)
  fia = (time_id * I + item_id).reshape(NW, QPW)

  mesh = plsc.VectorSubcoreMesh(core_axis_name="c", subcore_axis_name="s")
  qdp = jnp.pad(qd, ((0, 0), (0, IP - I)))
  simp = jnp.pad(item_sim_agg, ((0, 0), (0, IP - I)))
  f = pl.kernel(
      _sc_body,
      out_type=jax.ShapeDtypeStruct((B,), _f32),
      mesh=mesh,
      compiler_params=pltpu.CompilerParams(use_tc_tiling_on_sc=False,
                                           needs_layout_passes=False),
      scratch_types=[
          pltpu.VMEM((G, IP), _f32),
          pltpu.VMEM((G, IP), _f32),
          pltpu.VMEM((NCHUNK, G), _i32),
          pltpu.VMEM((NCHUNK, G), _i32),
          pltpu.VMEM((QPW,), _i32),
          pltpu.VMEM((QPW,), _f32),
          pltpu.VMEM((QPW,), _f32),
          pltpu.VMEM((CANDCAP,), _f32),
          pltpu.VMEM((CANDCAP,), _f32),
          pltpu.VMEM((512,), _i32),
          pltpu.VMEM((512,), _f32),
          pltpu.VMEM((512,), _f32),
          pltpu.SemaphoreType.DMA,
          pltpu.SemaphoreType.DMA,
          pltpu.SemaphoreType.DMA,
      ],
  )
  return f(simp, qdp, avgflat, rq, rs, fia)


# speedup vs baseline: 14.2950x; 1.0618x over previous
import jax
import jax.numpy as jnp
from jax import lax
from jax.experimental import pallas as pl
from jax.experimental.pallas import tpu as pltpu
from jax.experimental.pallas import tpu_sc as plsc

T, U, I = 16, 142, 4500
IP = 4608
K = 50
B = 4096
NW = 32
QPW = B // NW
G = 4
NCHUNK = QPW // G
L = 16
NB = IP // L
CANDCAP = 4608
MAXROUNDS = 24

_i32 = jnp.int32
_f32 = jnp.float32


def _iota():
  return lax.iota(_i32, L)


def _digit(v, base, scale):
  dv = jnp.clip((v - base) * scale, 0.0, 31.0)
  return dv.astype(_i32)


def _sc_body(sim_hbm, qd_hbm, avgflat_hbm, rq_hbm, rs_hbm,
             fia_hbm, out_hbm, simb, qdb, idxq, idxs, fi, avgi,
             preds, candv, candd, histc, hists, histn, sem_s0, sem_q0,
             sem_s1, sem_q1, sem_f):
  wid = lax.axis_index("s") * 2 + lax.axis_index("c")
  iot = _iota()
  zf = jnp.zeros((L,), _f32)
  zi = jnp.zeros((L,), _i32)
  ones_i = jnp.ones((L,), _i32)

  pltpu.sync_copy(rq_hbm.at[wid], idxq)
  pltpu.sync_copy(rs_hbm.at[wid], idxs)
  pltpu.sync_copy(fia_hbm.at[wid], fi)
  pltpu.async_copy(avgflat_hbm.at[fi], avgi, sem_f).wait()

  for h, z in ((histc, zi), (hists, zf), (histn, zf)):
    for j in range(512 // L):
      h[pl.ds(j * L, L)] = z

  def fold_hists():
    c0 = zi
    c1 = zi
    s0 = zf
    s1 = zf
    n0 = zf
    n1 = zf
    for lane in range(L):
      o = lane * 32
      idx0 = o + ((iot + lane) & 31)
      idx1 = o + ((iot + L + lane) & 31)
      c0 = c0 + plsc.load_gather(histc, [idx0])
      c1 = c1 + plsc.load_gather(histc, [idx1])
      s0 = s0 + plsc.load_gather(hists, [idx0])
      s1 = s1 + plsc.load_gather(hists, [idx1])
      n0 = n0 + plsc.load_gather(histn, [idx0])
      n1 = n1 + plsc.load_gather(histn, [idx1])
      plsc.store_scatter(histc, [idx0], zi)
      plsc.store_scatter(histc, [idx1], zi)
      plsc.store_scatter(hists, [idx0], zf)
      plsc.store_scatter(hists, [idx1], zf)
      plsc.store_scatter(histn, [idx0], zf)
      plsc.store_scatter(histn, [idx1], zf)
    return c0, c1, s0, s1, n0, n1

  def pick_bin(c0, c1, s0, s1, n0, n1, kk):
    t1 = jnp.sum(c1)
    rc0 = lax.rev(plsc.cumsum(lax.rev(c0, (0,))), (0,)) + t1
    rc1 = lax.rev(plsc.cumsum(lax.rev(c1, (0,))), (0,))
    b0 = jnp.max(jnp.where(rc0 >= kk, iot, -1))
    b1 = jnp.max(jnp.where(rc1 >= kk, iot + L, -1))
    bstar = jnp.maximum(b0, b1)
    gt0 = iot > bstar
    gt1 = (iot + L) > bstar
    c_gt = jnp.sum(jnp.where(gt0, c0, 0)) + jnp.sum(jnp.where(gt1, c1, 0))
    s_d = jnp.sum(jnp.where(gt0, s0, 0.0)) + jnp.sum(jnp.where(gt1, s1, 0.0))
    n_d = jnp.sum(jnp.where(gt0, n0, 0.0)) + jnp.sum(jnp.where(gt1, n1, 0.0))
    eq0 = iot == bstar
    eq1 = (iot + L) == bstar
    bincnt = (jnp.sum(jnp.where(eq0, c0, 0)) + jnp.sum(jnp.where(eq1, c1, 0)))
    return bstar, c_gt, s_d, n_d, bincnt

  def process_query(q, simr, qdr):
    def row_vd(j):
      o = pl.multiple_of(j * L, L)
      s = simr[pl.ds(o, L)]
      d = qdr[pl.ds(o, L)]
      v = jnp.where(d == d, s, 0.0)
      return v, d

    def hist_scatter(v, d, mask):
      dig = _digit(v, 0.0, 32.0)
      hidx = iot * 32 + ((dig + iot) & 31)
      plsc.addupdate_scatter(histc, [hidx], ones_i, mask=mask)
      plsc.addupdate_scatter(hists, [hidx], v, mask=mask)
      plsc.addupdate_scatter(histn, [hidx], v * d, mask=mask)

    @plsc.parallel_loop(0, NB, unroll=8)
    def p1(j):
      v, d = row_vd(j)
      hist_scatter(v, d, v > 0.0)

    c0, c1, s0, s1, n0, n1 = fold_hists()
    count_pos = jnp.sum(c0) + jnp.sum(c1)

    def direct_path():
      return jnp.sum(s0) + jnp.sum(s1), jnp.sum(n0) + jnp.sum(n1)

    def select_path():
      bstar, c_gt, s_d, n_d, bincnt = pick_bin(c0, c1, s0, s1, n0, n1, K)

      def compact_step(v, d, m, carry):
        off, cmin_v, cmax_v = carry
        mi = m.astype(_i32)
        posn = off + plsc.cumsum(mi) - mi
        plsc.store_scatter(candv, [posn], v, mask=m)
        plsc.store_scatter(candd, [posn], d, mask=m)
        off = off + plsc.all_reduce_population_count(m)
        cmin_v = jnp.minimum(cmin_v, jnp.where(m, v, jnp.float32(3e38)))
        cmax_v = jnp.maximum(cmax_v, jnp.where(m, v, jnp.float32(-3e38)))
        return off, cmin_v, cmax_v

      def compact_init():
        return (zi, jnp.full((L,), 3e38, _f32), jnp.full((L,), -3e38, _f32))

      @plsc.parallel_loop(0, NB, unroll=4, carry=compact_init())
      def rcompact(j, carry):
        v, d = row_vd(j)
        m = (v > 0.0) & (_digit(v, 0.0, 32.0) == bstar)
        return compact_step(v, d, m, carry)

      off, cmin_v, cmax_v = rcompact
      cmin = jnp.min(cmin_v)
      cmax = jnp.max(cmax_v)
      kk = K - c_gt
      ncand = bincnt
      state = jnp.where(
          ncand <= L, _i32(1),
          jnp.where((kk == ncand) | (cmin == cmax), _i32(2), _i32(0)))

      def cond(carry):
        return carry[-1] == 0

      def body(carry):
        base, scale, kk, ncand, s_ab, n_ab, rnd, _ = carry
        trips = (ncand + (L - 1)) // L

        def cand_vd(j):
          o = pl.multiple_of(j * L, L)
          return candv[pl.ds(o, L)], candd[pl.ds(o, L)], (o + iot) < ncand

        @plsc.parallel_loop(0, trips, unroll=2)
        def chist(j):
          v, d, valid = cand_vd(j)
          hidx = iot * 32 + ((_digit(v, base, scale) + iot) & 31)
          plsc.addupdate_scatter(histc, [hidx], ones_i, mask=valid)
          plsc.addupdate_scatter(hists, [hidx], v, mask=valid)
          plsc.addupdate_scatter(histn, [hidx], v * d, mask=valid)
        c0, c1, s0, s1, n0, n1 = fold_hists()
        bstar, c_gt, s_d, n_d, bincnt = pick_bin(c0, c1, s0, s1, n0, n1, kk)

        @plsc.parallel_loop(0, trips, unroll=2, carry=compact_init())
        def ccompact(j, carry):
          v, d, valid = cand_vd(j)
          m = valid & (_digit(v, base, scale) == bstar)
          return compact_step(v, d, m, carry)

        off, cmin_v, cmax_v = ccompact
        cmin = jnp.min(cmin_v)
        cmax = jnp.max(cmax_v)
        kk2 = kk - c_gt
        ncand2 = bincnt
        rnd2 = rnd + 1
        state2 = jnp.where(
            ncand2 <= L, _i32(1),
            jnp.where((kk2 == ncand2) | (cmin == cmax) | (rnd2 >= MAXROUNDS),
                      _i32(2), _i32(0)))
        base2 = cmin
        denom = jnp.maximum(cmax - cmin, jnp.float32(1e-38))
        scale2 = (jnp.full((L,), 32.0, _f32) / jnp.full((L,), denom, _f32))[0]
        return (base2, scale2, kk2, ncand2, s_ab + s_d, n_ab + n_d, rnd2,
                state2)

      base, scale, kk, ncand, s_ab, n_ab, rnd, state = lax.while_loop(
          cond, body,
          (jnp.float32(0.0), jnp.float32(32.0), kk, ncand, s_d, n_d,
           _i32(1), state))

      def term_sort():
        v = candv[pl.ds(0, L)]
        d = candd[pl.ds(0, L)]
        valid = iot < ncand
        ve = jnp.where(valid, v, -1.0)
        de = jnp.where(valid, d, 0.0)
        sk, _ = plsc.sort_key_val(ve, de, descending=True)
        theta = jnp.sum(jnp.where(iot == kk - 1, sk, 0.0))
        cntgt = jnp.sum(jnp.where(ve > theta, 1, 0))
        mfill = kk - cntgt
        tie = ve == theta
        pref = plsc.cumsum(tie.astype(_i32))
        sel = (ve > theta) | (tie & (pref <= mfill))
        s_t = jnp.sum(jnp.where(sel, ve, 0.0))
        n_t = jnp.sum(jnp.where(sel, ve * de, 0.0))
        return s_t, n_t

      def term_first():
        trips = (kk + (L - 1)) // L

        def tf(j, carry):
          sa, na = carry
          o = pl.multiple_of(j * L, L)
          v = candv[pl.ds(o, L)]
          d = candd[pl.ds(o, L)]
          valid = (o + iot) < kk
          sa = sa + jnp.where(valid, v, 0.0)
          na = na + jnp.where(valid, v * d, 0.0)
          return sa, na

        sa, na = lax.fori_loop(0, trips, tf, (zf, zf))
        return jnp.sum(sa), jnp.sum(na)

      s_t, n_t = lax.cond(state == 1, term_sort, term_first)
      return s_ab + s_t, n_ab + n_t

    S, N = lax.cond(count_pos <= K, direct_path, select_path)
    av = plsc.load_gather(avgi, [jnp.full((L,), q, _i32)])
    ratio = (jnp.full((L,), N, _f32)
             / jnp.full((L,), S + jnp.float32(1e-8), _f32))[0]
    pred = av[0] + ratio
    plsc.store_scatter(preds, [jnp.full((L,), q, _i32)],
                       jnp.full((L,), pred, _f32), mask=iot == 0)

  def issue(c, buf_s, buf_q, sem_s, sem_q):
    pltpu.async_copy(sim_hbm.at[idxs.at[c]], buf_s, sem_s)
    pltpu.async_copy(qd_hbm.at[idxq.at[c]], buf_q, sem_q)

  def drain(c, buf_s, buf_q, sem_s, sem_q):
    pltpu.make_async_copy(sim_hbm.at[idxs.at[c]], buf_s, sem_s).wait()
    pltpu.make_async_copy(qd_hbm.at[idxq.at[c]], buf_q, sem_q).wait()

  def do_queries(c, buf_s, buf_q):
    def per_q(g, _):
      process_query(c * G + g, buf_s.at[g], buf_q.at[g])
      return 0

    lax.fori_loop(0, G, per_q, 0)

  issue(0, simb.at[0], qdb.at[0], sem_s0, sem_q0)

  def chunk2(i, _):
    ca = 2 * i
    cb = 2 * i + 1
    issue(cb, simb.at[1], qdb.at[1], sem_s1, sem_q1)
    drain(ca, simb.at[0], qdb.at[0], sem_s0, sem_q0)
    do_queries(ca, simb.at[0], qdb.at[0])

    @pl.when(i + 1 < NCHUNK // 2)
    def _():
      issue(ca + 2, simb.at[0], qdb.at[0], sem_s0, sem_q0)

    drain(cb, simb.at[1], qdb.at[1], sem_s1, sem_q1)
    do_queries(cb, simb.at[1], qdb.at[1])
    return 0

  lax.fori_loop(0, NCHUNK // 2, chunk2, 0)
  pltpu.sync_copy(preds, out_hbm.at[pl.ds(pl.multiple_of(wid * QPW, QPW),
                                          QPW)])


def kernel(qos, item_avg, item_sim_agg, user_id, item_id, time_id):
  qd = jnp.where(qos > 0.0, qos - item_avg[:, None, :],
                 jnp.float32(jnp.nan)).reshape(T * U, I)
  avgflat = item_avg.reshape(T * I)
  user_id = user_id.astype(_i32)
  item_id = item_id.astype(_i32)
  time_id = time_id.astype(_i32)
  rq = (time_id * U + user_id).reshape(NW, NCHUNK, G)
  rs = item_id.reshape(NW, NCHUNK, G)
  fia = (time_id * I + item_id).reshape(NW, QPW)

  mesh = plsc.VectorSubcoreMesh(core_axis_name="c", subcore_axis_name="s")
  qdp = jnp.pad(qd, ((0, 0), (0, IP - I)))
  simp = jnp.pad(item_sim_agg, ((0, 0), (0, IP - I)))
  f = pl.kernel(
      _sc_body,
      out_type=jax.ShapeDtypeStruct((B,), _f32),
      mesh=mesh,
      compiler_params=pltpu.CompilerParams(use_tc_tiling_on_sc=False,
                                           needs_layout_passes=False),
      scratch_types=[
          pltpu.VMEM((2, G, IP), _f32),
          pltpu.VMEM((2, G, IP), _f32),
          pltpu.VMEM((NCHUNK, G), _i32),
          pltpu.VMEM((NCHUNK, G), _i32),
          pltpu.VMEM((QPW,), _i32),
          pltpu.VMEM((QPW,), _f32),
          pltpu.VMEM((QPW,), _f32),
          pltpu.VMEM((CANDCAP,), _f32),
          pltpu.VMEM((CANDCAP,), _f32),
          pltpu.VMEM((512,), _i32),
          pltpu.VMEM((512,), _f32),
          pltpu.VMEM((512,), _f32),
          pltpu.SemaphoreType.DMA,
          pltpu.SemaphoreType.DMA,
          pltpu.SemaphoreType.DMA,
          pltpu.SemaphoreType.DMA,
          pltpu.SemaphoreType.DMA,
      ],
  )
  return f(simp, qdp, avgflat, rq, rs, fia)
